# Initial kernel scaffold; baseline (speedup 1.0000x reference)
import jax
import jax.numpy as jnp
from jax.experimental import pallas as pl


def kernel(entity_emb, user_emb, relation_emb, edge_index, edge_type, inter_user, inter_item, inter_edge_w):
    raise NotImplementedError("write your pallas kernel here")


# speedup vs baseline: 2.3839x; 2.3839x over previous
import jax
import jax.numpy as jnp
from jax.experimental import pallas as pl

N_ITEMS = 30000
HOPS = 2


def kernel(entity_emb, user_emb, relation_emb, edge_index, edge_type, inter_user, inter_item, inter_edge_w):
    n_entities = entity_emb.shape[0]
    n_users = user_emb.shape[0]
    D = entity_emb.shape[1]
    head = edge_index[0]
    tail = edge_index[1]

    ent = entity_emb
    ent_res = entity_emb
    for _ in range(HOPS):
        r = relation_emb[edge_type]
        h = ent[head]
        t = ent[tail]
        s = jnp.sum(h * r * t, axis=-1) / jnp.sqrt(jnp.asarray(D, jnp.float32))
        es = jnp.exp(s)
        agg = jax.ops.segment_sum(es[:, None] * (t * r), head, num_segments=n_entities)
        ent = agg / (jnp.linalg.norm(agg, axis=-1, keepdims=True) + 1e-8)
        ent_res = ent_res + ent

    u = user_emb
    i = entity_emb[:N_ITEMS]
    u_res = u
    i_res = i
    for _ in range(HOPS):
        u_new = jax.ops.segment_sum(inter_edge_w[:, None] * i[inter_item], inter_user, num_segments=n_users)
        i_new = jax.ops.segment_sum(inter_edge_w[:, None] * u[inter_user], inter_item, num_segments=N_ITEMS)
        u, i = u_new, i_new
        u_res = u_res + u
        i_res = i_res + i

    return ent_res, u_res, i_res


# speedup vs baseline: 4.5735x; 1.9185x over previous
import functools

import jax
import jax.numpy as jnp
from jax import lax
from jax.experimental import pallas as pl
from jax.experimental.pallas import tpu as pltpu
from jax.experimental.pallas import tpu_sc as plsc

NC, NS = 2, 16
D = 64
NQ = D // 16

N_ENT = 100000
KG_NR = 12800
KG_NRANGE = 8
N_ENT_PAD = KG_NR * KG_NRANGE
KG_R_PER_SC = KG_NRANGE // NC
KG_ROWS_TILE = KG_NR // NS
E_KG = 1000000
E_KG_PAD = 1 << 20
KG_SP = E_KG_PAD // NS
CH = 4096
KG_NCH = KG_SP // CH
GRP = (CH + 16) // 16

N_USR = 50000
N_ITM = 30000
U_NR = 25600
U_PAD = U_NR * NC
U_ROWS_TILE = U_NR // NS
I_NR = 15360
I_PAD = I_NR * NC
I_ROWS_TILE = I_NR // NS
E_UI = 500000
E_UI_PAD = 1 << 19
UI_SP = E_UI_PAD // NS
UI_NCH = UI_SP // CH

_SC_PARAMS = pltpu.CompilerParams(
    use_tc_tiling_on_sc=False, needs_layout_passes=False)
_MESH = plsc.VectorSubcoreMesh(core_axis_name="c", subcore_axis_name="s")


def _zero_fill(buf):
    zero16 = jnp.zeros((16,), jnp.float32)
    for j in range(16):
        for q in range(NQ):
            buf[j, pl.ds(q * 16, 16)] = zero16


def _zero_acc(acc, zbuf, sid, rows_tile):
    def zb(z, _):
        pltpu.sync_copy(zbuf, acc.at[pl.ds(sid * rows_tile + z * 16, 16)])
        return 0
    lax.fori_loop(0, rows_tile // 16, zb, 0)


@functools.partial(
    pl.kernel, mesh=_MESH,
    out_type=jax.ShapeDtypeStruct((N_ENT_PAD, D), jnp.float32),
    compiler_params=_SC_PARAMS,
    scratch_types=[
        pltpu.VMEM_SHARED((KG_NR, D), jnp.float32),
        pltpu.VMEM((8, D), jnp.float32),
        pltpu.VMEM((CH,), jnp.int32),
        pltpu.VMEM((CH,), jnp.int32),
        pltpu.VMEM((CH,), jnp.int32),
        pltpu.VMEM((GRP, 16), jnp.int32),
        pltpu.VMEM((GRP, 16), jnp.int32),
        pltpu.VMEM((GRP, 16), jnp.int32),
        pltpu.VMEM((CH + 16,), jnp.int32),
        pltpu.VMEM((16, D), jnp.float32),
        pltpu.VMEM((16, D), jnp.float32),
        pltpu.VMEM((16, D), jnp.float32),
        pltpu.VMEM((16, D), jnp.float32),
        pltpu.SemaphoreType.DMA,
        pltpu.SemaphoreType.DMA,
    ])
def _kg_pass(ent_hbm, rel_hbm, head_hbm, tail_hbm, type_hbm, agg_hbm,
             acc, rtab, hc, tc_, yc, chabs, cdst, ctail, ctype,
             hrows, trows, orows, zbuf, semh, semt):
    cid = lax.axis_index("c")
    sid = lax.axis_index("s")
    iota = lax.iota(jnp.int32, 16)
    pltpu.sync_copy(rel_hbm, rtab)
    _zero_fill(zbuf)

    def range_body(rr, _):
        lo = (rr * NC + cid) * KG_NR
        _zero_acc(acc, zbuf, sid, KG_ROWS_TILE)
        plsc.subcore_barrier()

        def chunk_body(c, _):
            base = sid * KG_SP + c * CH
            pltpu.sync_copy(head_hbm.at[pl.ds(base, CH)], hc)
            pltpu.sync_copy(tail_hbm.at[pl.ds(base, CH)], tc_)
            pltpu.sync_copy(type_hbm.at[pl.ds(base, CH)], yc)

            def scan(g, off):
                hv = hc[pl.ds(g * 16, 16)]
                tv = tc_[pl.ds(g * 16, 16)]
                yv = yc[pl.ds(g * 16, 16)]
                m = (hv >= lo) & (hv < lo + KG_NR)
                pos = off + plsc.cumsum(m.astype(jnp.int32)) - 1
                pr = lax.shift_right_logical(pos, 4)
                pc = pos & 15
                plsc.store_scatter(chabs, [pr, pc], hv, mask=m)
                plsc.store_scatter(cdst, [pr, pc], hv - lo, mask=m)
                plsc.store_scatter(ctail, [pr, pc], tv, mask=m)
                plsc.store_scatter(ctype, [pos], yv, mask=m)
                return off + jnp.sum(m.astype(jnp.int32))

            n = lax.fori_loop(0, CH // 16, scan, jnp.int32(0))
            npos = n + iota
            nr_ = lax.shift_right_logical(npos, 4)
            ncl = npos & 15
            pad_row = jnp.full((16,), N_ENT, jnp.int32)
            zero_i = jnp.zeros((16,), jnp.int32)
            plsc.store_scatter(chabs, [nr_, ncl], pad_row)
            plsc.store_scatter(cdst, [nr_, ncl], zero_i)
            plsc.store_scatter(ctail, [nr_, ncl], pad_row)
            plsc.store_scatter(ctype, [npos], zero_i)
            ng = lax.shift_right_logical(n + 15, 4)

            def batch(g, _):
                cph = pltpu.async_copy(ent_hbm.at[chabs.at[g]], hrows, semh)
                cpt = pltpu.async_copy(ent_hbm.at[ctail.at[g]], trows, semt)
                cph.wait()
                cpt.wait()
                gbase = g * 16
                for j in range(16):
                    ty = plsc.load_gather(
                        ctype, [gbase + j + jnp.zeros((16,), jnp.int32)])
                    cqs = []
                    sacc = None
                    for q in range(NQ):
                        rq = plsc.load_gather(rtab, [ty, iota + (q * 16)])
                        hq = hrows[j, pl.ds(q * 16, 16)]
                        tq = trows[j, pl.ds(q * 16, 16)]
                        cq = tq * rq
                        cqs.append(cq)
                        p = hq * cq
                        sacc = p if sacc is None else sacc + p
                    s = jnp.sum(sacc) * 0.125
                    esv = jnp.exp(jnp.full((16,), s, jnp.float32))
                    for q in range(NQ):
                        orows[j, pl.ds(q * 16, 16)] = esv * cqs[q]
                pltpu.sync_copy(orows, acc.at[cdst.at[g]], add=True)
                return 0

            lax.fori_loop(0, ng, batch, 0)
            return 0

        lax.fori_loop(0, KG_NCH, chunk_body, 0)
        plsc.subcore_barrier()
        pltpu.sync_copy(
            acc.at[pl.ds(sid * KG_ROWS_TILE, KG_ROWS_TILE)],
            agg_hbm.at[pl.ds(lo + sid * KG_ROWS_TILE, KG_ROWS_TILE)])
        plsc.subcore_barrier()
        return 0

    lax.fori_loop(0, KG_R_PER_SC, range_body, 0)


@functools.partial(
    pl.kernel, mesh=_MESH,
    out_type=(jax.ShapeDtypeStruct((U_PAD, D), jnp.float32),
              jax.ShapeDtypeStruct((I_PAD, D), jnp.float32)),
    compiler_params=_SC_PARAMS,
    scratch_types=[
        pltpu.VMEM_SHARED((U_NR, D), jnp.float32),
        pltpu.VMEM((CH,), jnp.int32),
        pltpu.VMEM((CH,), jnp.int32),
        pltpu.VMEM((CH,), jnp.float32),
        pltpu.VMEM((GRP, 16), jnp.int32),
        pltpu.VMEM((GRP, 16), jnp.int32),
        pltpu.VMEM((CH + 16,), jnp.float32),
        pltpu.VMEM((16, D), jnp.float32),
        pltpu.VMEM((16, D), jnp.float32),
        pltpu.VMEM((16, D), jnp.float32),
        pltpu.SemaphoreType.DMA,
    ])
def _ui_pass(usr_hbm, itm_hbm, iu_hbm, ii_hbm, w_hbm, uo_hbm, io_hbm,
             acc, dc, sc_, wc, cdst, csrc, cw, rows, orows, zbuf, semg):
    cid = lax.axis_index("c")
    sid = lax.axis_index("s")
    iota = lax.iota(jnp.int32, 16)
    _zero_fill(zbuf)

    def side(dst_hbm_, src_hbm_, tab_hbm_, out_hbm_, nr_side, rows_tile):
        lo = cid * nr_side
        _zero_acc(acc, zbuf, sid, rows_tile)
        plsc.subcore_barrier()

        def chunk_body(c, _):
            base = sid * UI_SP + c * CH
            pltpu.sync_copy(dst_hbm_.at[pl.ds(base, CH)], dc)
            pltpu.sync_copy(src_hbm_.at[pl.ds(base, CH)], sc_)
            pltpu.sync_copy(w_hbm.at[pl.ds(base, CH)], wc)

            def scan(g, off):
                dv = dc[pl.ds(g * 16, 16)]
                sv = sc_[pl.ds(g * 16, 16)]
                wv = wc[pl.ds(g * 16, 16)]
                m = (dv >= lo) & (dv < lo + nr_side)
                pos = off + plsc.cumsum(m.astype(jnp.int32)) - 1
                pr = lax.shift_right_logical(pos, 4)
                pc = pos & 15
                plsc.store_scatter(cdst, [pr, pc], dv - lo, mask=m)
                plsc.store_scatter(csrc, [pr, pc], sv, mask=m)
                plsc.store_scatter(cw, [pos], wv, mask=m)
                return off + jnp.sum(m.astype(jnp.int32))

            n = lax.fori_loop(0, CH // 16, scan, jnp.int32(0))
            npos = n + iota
            nr_ = lax.shift_right_logical(npos, 4)
            ncl = npos & 15
            zero_i = jnp.zeros((16,), jnp.int32)
            plsc.store_scatter(cdst, [nr_, ncl], zero_i)
            plsc.store_scatter(csrc, [nr_, ncl], zero_i)
            plsc.store_scatter(cw, [npos], jnp.zeros((16,), jnp.float32))
            ng = lax.shift_right_logical(n + 15, 4)

            def batch(g, _):
                pltpu.async_copy(tab_hbm_.at[csrc.at[g]], rows, semg).wait()
                gbase = g * 16
                for j in range(16):
                    wsp = plsc.load_gather(
                        cw, [gbase + j + jnp.zeros((16,), jnp.int32)])
                    for q in range(NQ):
                        orows[j, pl.ds(q * 16, 16)] = (
                            wsp * rows[j, pl.ds(q * 16, 16)])
                pltpu.sync_copy(orows, acc.at[cdst.at[g]], add=True)
                return 0

            lax.fori_loop(0, ng, batch, 0)
            return 0

        lax.fori_loop(0, UI_NCH, chunk_body, 0)
        plsc.subcore_barrier()
        pltpu.sync_copy(
            acc.at[pl.ds(sid * rows_tile, rows_tile)],
            out_hbm_.at[pl.ds(lo + sid * rows_tile, rows_tile)])
        plsc.subcore_barrier()

    side(iu_hbm, ii_hbm, itm_hbm, uo_hbm, U_NR, U_ROWS_TILE)
    side(ii_hbm, iu_hbm, usr_hbm, io_hbm, I_NR, I_ROWS_TILE)


def _norm_body(agg_ref, resin_ref, ent_ref, resout_ref):
    x = agg_ref[...]
    ss = jnp.sum(x * x, axis=1, keepdims=True)
    ent = x / (jnp.sqrt(ss) + 1e-8)
    ent_ref[...] = ent
    resout_ref[...] = resin_ref[...] + ent


_NORM_BLK = 1024
_norm_call = pl.pallas_call(
    _norm_body,
    grid=(N_ENT_PAD // _NORM_BLK,),
    in_specs=[pl.BlockSpec((_NORM_BLK, D), lambda i: (i, 0))] * 2,
    out_specs=[pl.BlockSpec((_NORM_BLK, D), lambda i: (i, 0))] * 2,
    out_shape=(jax.ShapeDtypeStruct((N_ENT_PAD, D), jnp.float32),
               jax.ShapeDtypeStruct((N_ENT_PAD, D), jnp.float32)),
)


def _add3_body(a_ref, b_ref, c_ref, o_ref):
    o_ref[...] = a_ref[...] + b_ref[...] + c_ref[...]


def _add3(a, b, c, blk):
    n = a.shape[0]
    return pl.pallas_call(
        _add3_body,
        grid=(n // blk,),
        in_specs=[pl.BlockSpec((blk, D), lambda i: (i, 0))] * 3,
        out_specs=pl.BlockSpec((blk, D), lambda i: (i, 0)),
        out_shape=jax.ShapeDtypeStruct((n, D), jnp.float32),
    )(a, b, c)


def kernel(entity_emb, user_emb, relation_emb, edge_index, edge_type,
           inter_user, inter_item, inter_edge_w):
    f32 = jnp.float32
    i32 = jnp.int32
    head = edge_index[0].astype(i32)
    tail = edge_index[1].astype(i32)
    etype = edge_type.astype(i32)

    ent0 = jnp.zeros((N_ENT_PAD, D), f32).at[:N_ENT].set(entity_emb)
    headp = jnp.full((E_KG_PAD,), -1, i32).at[:E_KG].set(head)
    tailp = jnp.zeros((E_KG_PAD,), i32).at[:E_KG].set(tail)
    typep = jnp.zeros((E_KG_PAD,), i32).at[:E_KG].set(etype)
    iup = jnp.full((E_UI_PAD,), -1, i32).at[:E_UI].set(inter_user.astype(i32))
    iip = jnp.full((E_UI_PAD,), -1, i32).at[:E_UI].set(inter_item.astype(i32))
    wp = jnp.zeros((E_UI_PAD,), f32).at[:E_UI].set(inter_edge_w)

    agg1 = _kg_pass(ent0, relation_emb, headp, tailp, typep)
    ent1, res1 = _norm_call(agg1, ent0)
    agg2 = _kg_pass(ent1, relation_emb, headp, tailp, typep)
    _, res2 = _norm_call(agg2, res1)
    ent_res = res2[:N_ENT]

    u0 = user_emb
    i0 = entity_emb[:N_ITM]
    u1p, i1p = _ui_pass(u0, i0, iup, iip, wp)
    u1 = u1p[:N_USR]
    i1 = i1p[:N_ITM]
    u2p, i2p = _ui_pass(u1, i1, iup, iip, wp)
    u_res = _add3(u0, u1, u2p[:N_USR], 1000)
    i_res = _add3(i0, i1, i2p[:N_ITM], 1000)
    return ent_res, u_res, i_res


# speedup vs baseline: 6.6520x; 1.4545x over previous
import functools

import jax
import jax.numpy as jnp
from jax import lax
from jax.experimental import pallas as pl
from jax.experimental.pallas import tpu as pltpu
from jax.experimental.pallas import tpu_sc as plsc

NC, NS = 2, 16
D = 64
NQ = D // 16

N_ENT = 100000
KG_NR = 12800
KG_NRANGE = 8
N_ENT_PAD = KG_NR * KG_NRANGE
KG_R_PER_SC = KG_NRANGE // NC
KG_ROWS_TILE = KG_NR // NS
E_KG = 1000000
E_KG_PAD = 1 << 20
KG_SP = E_KG_PAD // NS
CH = 4096
KG_NCH = KG_SP // CH
GRP = (CH + 16) // 16

N_USR = 50000
N_ITM = 30000
U_NR = 25600
U_PAD = U_NR * NC
U_ROWS_TILE = U_NR // NS
I_NR = 15360
I_PAD = I_NR * NC
I_ROWS_TILE = I_NR // NS
E_UI = 500000
E_UI_PAD = 1 << 19
UI_SP = E_UI_PAD // NS
CHU = 2048
UI_NCH = UI_SP // CHU
GRPU = (CHU + 16) // 16

_SC_PARAMS = pltpu.CompilerParams(
    use_tc_tiling_on_sc=False, needs_layout_passes=False)
_MESH = plsc.VectorSubcoreMesh(core_axis_name="c", subcore_axis_name="s")


def _zero_fill(buf):
    zero16 = jnp.zeros((16,), jnp.float32)
    for j in range(16):
        for q in range(NQ):
            buf[j, pl.ds(q * 16, 16)] = zero16


def _zero_acc(acc, zbuf, sid, rows_tile):
    def zb(z, _):
        pltpu.sync_copy(zbuf, acc.at[pl.ds(sid * rows_tile + z * 16, 16)])
        return 0
    lax.fori_loop(0, rows_tile // 16, zb, 0)


@functools.partial(
    pl.kernel, mesh=_MESH,
    out_type=jax.ShapeDtypeStruct((N_ENT_PAD, D), jnp.float32),
    compiler_params=_SC_PARAMS,
    scratch_types=[
        pltpu.VMEM_SHARED((KG_NR, D), jnp.float32),
        pltpu.VMEM((8, D), jnp.float32),
        pltpu.VMEM((CH,), jnp.int32),
        pltpu.VMEM((CH,), jnp.int32),
        pltpu.VMEM((CH,), jnp.int32),
        pltpu.VMEM((GRP, 16), jnp.int32),
        pltpu.VMEM((GRP, 16), jnp.int32),
        pltpu.VMEM((GRP, 16), jnp.int32),
        pltpu.VMEM((CH + 16,), jnp.int32),
        pltpu.VMEM((4 * 16, D), jnp.float32),
        pltpu.VMEM((4 * 16, D), jnp.float32),
        pltpu.VMEM((16, D), jnp.float32),
        pltpu.VMEM((16, D), jnp.float32),
        pltpu.SemaphoreType.DMA((4,)),
        pltpu.SemaphoreType.DMA((4,)),
    ])
def _kg_pass(ent_hbm, rel_hbm, head_hbm, tail_hbm, type_hbm, agg_hbm,
             acc, rtab, hc, tc_, yc, chabs, cdst, ctail, ctype,
             hrows, trows, orows, zbuf, semh, semt):
    cid = lax.axis_index("c")
    sid = lax.axis_index("s")
    iota = lax.iota(jnp.int32, 16)
    pltpu.sync_copy(rel_hbm, rtab)
    _zero_fill(zbuf)

    def range_body(rr, _):
        lo = (rr * NC + cid) * KG_NR
        _zero_acc(acc, zbuf, sid, KG_ROWS_TILE)
        plsc.subcore_barrier()

        def chunk_body(c, _):
            base = sid * KG_SP + c * CH
            pltpu.sync_copy(head_hbm.at[pl.ds(base, CH)], hc)
            pltpu.sync_copy(tail_hbm.at[pl.ds(base, CH)], tc_)
            pltpu.sync_copy(type_hbm.at[pl.ds(base, CH)], yc)

            def scan(g, off):
                hv = hc[pl.ds(g * 16, 16)]
                tv = tc_[pl.ds(g * 16, 16)]
                yv = yc[pl.ds(g * 16, 16)]
                m = (hv >= lo) & (hv < lo + KG_NR)
                pos = off + plsc.cumsum(m.astype(jnp.int32)) - 1
                pr = lax.shift_right_logical(pos, 4)
                pc = pos & 15
                plsc.store_scatter(chabs, [pr, pc], hv, mask=m)
                plsc.store_scatter(cdst, [pr, pc], hv - lo, mask=m)
                plsc.store_scatter(ctail, [pr, pc], tv, mask=m)
                plsc.store_scatter(ctype, [pos], yv, mask=m)
                return off + jnp.sum(m.astype(jnp.int32))

            n = lax.fori_loop(0, CH // 16, scan, jnp.int32(0))
            npos = n + iota
            nr_ = lax.shift_right_logical(npos, 4)
            ncl = npos & 15
            pad_row = jnp.full((16,), N_ENT, jnp.int32)
            zero_i = jnp.zeros((16,), jnp.int32)
            plsc.store_scatter(chabs, [nr_, ncl], pad_row)
            plsc.store_scatter(cdst, [nr_, ncl], zero_i)
            plsc.store_scatter(ctail, [nr_, ncl], pad_row)
            plsc.store_scatter(ctype, [npos], zero_i)
            ng = lax.shift_right_logical(n + 15, 4)

            def fire(g, b):
                pltpu.async_copy(ent_hbm.at[chabs.at[g]],
                                 hrows.at[pl.ds(b * 16, 16)], semh.at[b])
                pltpu.async_copy(ent_hbm.at[ctail.at[g]],
                                 trows.at[pl.ds(b * 16, 16)], semt.at[b])

            def drain(g, b):
                pltpu.make_async_copy(ent_hbm.at[chabs.at[g]],
                                      hrows.at[pl.ds(b * 16, 16)],
                                      semh.at[b]).wait()
                pltpu.make_async_copy(ent_hbm.at[ctail.at[g]],
                                      trows.at[pl.ds(b * 16, 16)],
                                      semt.at[b]).wait()

            for b0 in range(3):
                @pl.when(b0 < ng)
                def _(b0=b0):
                    fire(b0, b0)

            def quad(qi, _):
                gp = qi * 4
                for b in range(4):
                    g = gp + b

                    @pl.when(g < ng)
                    def _(g=g, b=b):
                        drain(g, b)

                        @pl.when(g + 3 < ng)
                        def _():
                            fire(g + 3, (b + 3) % 4)

                        gbase = g * 16
                        jbase = b * 16
                        for j in range(16):
                            ty = plsc.load_gather(
                                ctype,
                                [gbase + j + jnp.zeros((16,), jnp.int32)])
                            cqs = []
                            sacc = None
                            for q in range(NQ):
                                rq = plsc.load_gather(rtab,
                                                      [ty, iota + (q * 16)])
                                hq = hrows[jbase + j, pl.ds(q * 16, 16)]
                                tq = trows[jbase + j, pl.ds(q * 16, 16)]
                                cq = tq * rq
                                cqs.append(cq)
                                p = hq * cq
                                sacc = p if sacc is None else sacc + p
                            s = jnp.sum(sacc) * 0.125
                            esv = jnp.exp(jnp.full((16,), s, jnp.float32))
                            for q in range(NQ):
                                orows[j, pl.ds(q * 16, 16)] = esv * cqs[q]
                        pltpu.sync_copy(orows, acc.at[cdst.at[g]], add=True)
                return 0

            lax.fori_loop(0, lax.shift_right_logical(ng + 3, 2), quad, 0)
            return 0

        lax.fori_loop(0, KG_NCH, chunk_body, 0)
        plsc.subcore_barrier()
        pltpu.sync_copy(
            acc.at[pl.ds(sid * KG_ROWS_TILE, KG_ROWS_TILE)],
            agg_hbm.at[pl.ds(lo + sid * KG_ROWS_TILE, KG_ROWS_TILE)])
        plsc.subcore_barrier()
        return 0

    lax.fori_loop(0, KG_R_PER_SC, range_body, 0)


@functools.partial(
    pl.kernel, mesh=_MESH,
    out_type=(jax.ShapeDtypeStruct((U_PAD, D), jnp.float32),
              jax.ShapeDtypeStruct((I_PAD, D), jnp.float32)),
    compiler_params=_SC_PARAMS,
    scratch_types=[
        pltpu.VMEM_SHARED((U_NR, D), jnp.float32),
        pltpu.VMEM((CHU,), jnp.int32),
        pltpu.VMEM((CHU,), jnp.int32),
        pltpu.VMEM((CHU,), jnp.float32),
        pltpu.VMEM((GRPU, 16), jnp.int32),
        pltpu.VMEM((GRPU, 16), jnp.int32),
        pltpu.VMEM((CHU + 16,), jnp.float32),
        pltpu.VMEM((4 * 16, D), jnp.float32),
        pltpu.VMEM((16, D), jnp.float32),
        pltpu.VMEM((16, D), jnp.float32),
        pltpu.SemaphoreType.DMA((4,)),
    ])
def _ui_pass(usr_hbm, itm_hbm, iu_hbm, ii_hbm, w_hbm, uo_hbm, io_hbm,
             acc, dc, sc_, wc, cdst, csrc, cw, rows, orows, zbuf, semg):
    cid = lax.axis_index("c")
    sid = lax.axis_index("s")
    iota = lax.iota(jnp.int32, 16)
    _zero_fill(zbuf)

    def side(dst_hbm_, src_hbm_, tab_hbm_, out_hbm_, nr_side, rows_tile):
        lo = cid * nr_side
        _zero_acc(acc, zbuf, sid, rows_tile)
        plsc.subcore_barrier()

        def chunk_body(c, _):
            base = sid * UI_SP + c * CHU
            pltpu.sync_copy(dst_hbm_.at[pl.ds(base, CHU)], dc)
            pltpu.sync_copy(src_hbm_.at[pl.ds(base, CHU)], sc_)
            pltpu.sync_copy(w_hbm.at[pl.ds(base, CHU)], wc)

            def scan(g, off):
                dv = dc[pl.ds(g * 16, 16)]
                sv = sc_[pl.ds(g * 16, 16)]
                wv = wc[pl.ds(g * 16, 16)]
                m = (dv >= lo) & (dv < lo + nr_side)
                pos = off + plsc.cumsum(m.astype(jnp.int32)) - 1
                pr = lax.shift_right_logical(pos, 4)
                pc = pos & 15
                plsc.store_scatter(cdst, [pr, pc], dv - lo, mask=m)
                plsc.store_scatter(csrc, [pr, pc], sv, mask=m)
                plsc.store_scatter(cw, [pos], wv, mask=m)
                return off + jnp.sum(m.astype(jnp.int32))

            n = lax.fori_loop(0, CHU // 16, scan, jnp.int32(0))
            npos = n + iota
            nr_ = lax.shift_right_logical(npos, 4)
            ncl = npos & 15
            zero_i = jnp.zeros((16,), jnp.int32)
            plsc.store_scatter(cdst, [nr_, ncl], zero_i)
            plsc.store_scatter(csrc, [nr_, ncl], zero_i)
            plsc.store_scatter(cw, [npos], jnp.zeros((16,), jnp.float32))
            ng = lax.shift_right_logical(n + 15, 4)

            def fire(g, b):
                pltpu.async_copy(tab_hbm_.at[csrc.at[g]],
                                 rows.at[pl.ds(b * 16, 16)], semg.at[b])

            def drain(g, b):
                pltpu.make_async_copy(tab_hbm_.at[csrc.at[g]],
                                      rows.at[pl.ds(b * 16, 16)],
                                      semg.at[b]).wait()

            for b0 in range(3):
                @pl.when(b0 < ng)
                def _(b0=b0):
                    fire(b0, b0)

            def quad(qi, _):
                gp = qi * 4
                for b in range(4):
                    g = gp + b

                    @pl.when(g < ng)
                    def _(g=g, b=b):
                        drain(g, b)

                        @pl.when(g + 3 < ng)
                        def _():
                            fire(g + 3, (b + 3) % 4)

                        gbase = g * 16
                        jbase = b * 16
                        for j in range(16):
                            wsp = plsc.load_gather(
                                cw, [gbase + j + jnp.zeros((16,), jnp.int32)])
                            for q in range(NQ):
                                orows[j, pl.ds(q * 16, 16)] = (
                                    wsp * rows[jbase + j, pl.ds(q * 16, 16)])
                        pltpu.sync_copy(orows, acc.at[cdst.at[g]], add=True)
                return 0

            lax.fori_loop(0, lax.shift_right_logical(ng + 3, 2), quad, 0)
            return 0

        lax.fori_loop(0, UI_NCH, chunk_body, 0)
        plsc.subcore_barrier()
        pltpu.sync_copy(
            acc.at[pl.ds(sid * rows_tile, rows_tile)],
            out_hbm_.at[pl.ds(lo + sid * rows_tile, rows_tile)])
        plsc.subcore_barrier()

    side(iu_hbm, ii_hbm, itm_hbm, uo_hbm, U_NR, U_ROWS_TILE)
    side(ii_hbm, iu_hbm, usr_hbm, io_hbm, I_NR, I_ROWS_TILE)


def _norm_body(agg_ref, resin_ref, ent_ref, resout_ref):
    x = agg_ref[...]
    ss = jnp.sum(x * x, axis=1, keepdims=True)
    ent = x / (jnp.sqrt(ss) + 1e-8)
    ent_ref[...] = ent
    resout_ref[...] = resin_ref[...] + ent


_NORM_BLK = 1024
_norm_call = pl.pallas_call(
    _norm_body,
    grid=(N_ENT_PAD // _NORM_BLK,),
    in_specs=[pl.BlockSpec((_NORM_BLK, D), lambda i: (i, 0))] * 2,
    out_specs=[pl.BlockSpec((_NORM_BLK, D), lambda i: (i, 0))] * 2,
    out_shape=(jax.ShapeDtypeStruct((N_ENT_PAD, D), jnp.float32),
               jax.ShapeDtypeStruct((N_ENT_PAD, D), jnp.float32)),
)


def _add3_body(a_ref, b_ref, c_ref, o_ref):
    o_ref[...] = a_ref[...] + b_ref[...] + c_ref[...]


def _add3(a, b, c, blk):
    n = a.shape[0]
    return pl.pallas_call(
        _add3_body,
        grid=(n // blk,),
        in_specs=[pl.BlockSpec((blk, D), lambda i: (i, 0))] * 3,
        out_specs=pl.BlockSpec((blk, D), lambda i: (i, 0)),
        out_shape=jax.ShapeDtypeStruct((n, D), jnp.float32),
    )(a, b, c)


def kernel(entity_emb, user_emb, relation_emb, edge_index, edge_type,
           inter_user, inter_item, inter_edge_w):
    f32 = jnp.float32
    i32 = jnp.int32
    head = edge_index[0].astype(i32)
    tail = edge_index[1].astype(i32)
    etype = edge_type.astype(i32)

    ent0 = jnp.zeros((N_ENT_PAD, D), f32).at[:N_ENT].set(entity_emb)
    headp = jnp.full((E_KG_PAD,), -1, i32).at[:E_KG].set(head)
    tailp = jnp.zeros((E_KG_PAD,), i32).at[:E_KG].set(tail)
    typep = jnp.zeros((E_KG_PAD,), i32).at[:E_KG].set(etype)
    iup = jnp.full((E_UI_PAD,), -1, i32).at[:E_UI].set(inter_user.astype(i32))
    iip = jnp.full((E_UI_PAD,), -1, i32).at[:E_UI].set(inter_item.astype(i32))
    wp = jnp.zeros((E_UI_PAD,), f32).at[:E_UI].set(inter_edge_w)

    agg1 = _kg_pass(ent0, relation_emb, headp, tailp, typep)
    ent1, res1 = _norm_call(agg1, ent0)
    agg2 = _kg_pass(ent1, relation_emb, headp, tailp, typep)
    _, res2 = _norm_call(agg2, res1)
    ent_res = res2[:N_ENT]

    u0 = user_emb
    i0 = entity_emb[:N_ITM]
    u1p, i1p = _ui_pass(u0, i0, iup, iip, wp)
    u1 = u1p[:N_USR]
    i1 = i1p[:N_ITM]
    u2p, i2p = _ui_pass(u1, i1, iup, iip, wp)
    u_res = _add3(u0, u1, u2p[:N_USR], 1000)
    i_res = _add3(i0, i1, i2p[:N_ITM], 1000)
    return ent_res, u_res, i_res


# speedup vs baseline: 7.1487x; 1.0747x over previous
import functools

import jax
import jax.numpy as jnp
from jax import lax
from jax.experimental import pallas as pl
from jax.experimental.pallas import tpu as pltpu
from jax.experimental.pallas import tpu_sc as plsc

NC, NS = 2, 16
D = 64
NQ = D // 16

N_ENT = 100000
KG_NR = 12800
KG_NRANGE = 8
N_ENT_PAD = KG_NR * KG_NRANGE
KG_R_PER_SC = KG_NRANGE // NC
KG_ROWS_TILE = KG_NR // NS
E_KG = 1000000
E_KG_PAD = 1 << 20
KG_SP = E_KG_PAD // NS
CH = 4096
KG_NCH = KG_SP // CH
GRP = (CH + 16) // 16

N_USR = 50000
N_ITM = 30000
U_NR = 25600
U_PAD = U_NR * NC
U_ROWS_TILE = U_NR // NS
I_NR = 15360
I_PAD = I_NR * NC
I_ROWS_TILE = I_NR // NS
E_UI = 500000
E_UI_PAD = 1 << 19
UI_SP = E_UI_PAD // NS
CHU = 2048
UI_NCH = UI_SP // CHU
GRPU = (CHU + 16) // 16

_SC_PARAMS = pltpu.CompilerParams(
    use_tc_tiling_on_sc=False, needs_layout_passes=False)
_MESH = plsc.VectorSubcoreMesh(core_axis_name="c", subcore_axis_name="s")


def _zero_fill(buf):
    zero16 = jnp.zeros((16,), jnp.float32)
    for j in range(16):
        for q in range(NQ):
            buf[j, pl.ds(q * 16, 16)] = zero16


def _zero_acc(acc, zbuf, sid, rows_tile):
    def zb(z, _):
        pltpu.sync_copy(zbuf, acc.at[pl.ds(sid * rows_tile + z * 16, 16)])
        return 0
    lax.fori_loop(0, rows_tile // 16, zb, 0)


@functools.partial(
    pl.kernel, mesh=_MESH,
    out_type=jax.ShapeDtypeStruct((N_ENT_PAD, D), jnp.float32),
    compiler_params=_SC_PARAMS,
    scratch_types=[
        pltpu.VMEM_SHARED((KG_NR, D), jnp.float32),
        pltpu.VMEM((8, D), jnp.float32),
        pltpu.VMEM((CH,), jnp.int32),
        pltpu.VMEM((CH,), jnp.int32),
        pltpu.VMEM((CH,), jnp.int32),
        pltpu.VMEM((GRP, 16), jnp.int32),
        pltpu.VMEM((GRP, 16), jnp.int32),
        pltpu.VMEM((GRP, 16), jnp.int32),
        pltpu.VMEM((CH + 16,), jnp.int32),
        pltpu.VMEM((4 * 16, D), jnp.float32),
        pltpu.VMEM((4 * 16, D), jnp.float32),
        pltpu.VMEM((2 * 16, D), jnp.float32),
        pltpu.VMEM((16, D), jnp.float32),
        pltpu.SemaphoreType.DMA((4,)),
        pltpu.SemaphoreType.DMA((4,)),
        pltpu.SemaphoreType.DMA((2,)),
    ])
def _kg_pass(ent_hbm, rel_hbm, head_hbm, tail_hbm, type_hbm, agg_hbm,
             acc, rtab, hc, tc_, yc, chabs, cdst, ctail, ctype,
             hrows, trows, orows, zbuf, semh, semt, semo):
    cid = lax.axis_index("c")
    sid = lax.axis_index("s")
    iota = lax.iota(jnp.int32, 16)
    pltpu.sync_copy(rel_hbm, rtab)
    _zero_fill(zbuf)

    def range_body(rr, _):
        lo = (rr * NC + cid) * KG_NR
        _zero_acc(acc, zbuf, sid, KG_ROWS_TILE)
        plsc.subcore_barrier()

        def chunk_body(c, _):
            base = sid * KG_SP + c * CH
            c1 = pltpu.async_copy(head_hbm.at[pl.ds(base, CH)], hc, semh.at[0])
            c2 = pltpu.async_copy(tail_hbm.at[pl.ds(base, CH)], tc_, semt.at[0])
            c3 = pltpu.async_copy(type_hbm.at[pl.ds(base, CH)], yc, semt.at[1])
            c1.wait()
            c2.wait()
            c3.wait()

            def scan(g, off):
                hv = hc[pl.ds(g * 16, 16)]
                tv = tc_[pl.ds(g * 16, 16)]
                yv = yc[pl.ds(g * 16, 16)]
                m = (hv >= lo) & (hv < lo + KG_NR)
                pos = off + plsc.cumsum(m.astype(jnp.int32)) - 1
                pr = lax.shift_right_logical(pos, 4)
                pc = pos & 15
                plsc.store_scatter(chabs, [pr, pc], hv, mask=m)
                plsc.store_scatter(cdst, [pr, pc], hv - lo, mask=m)
                plsc.store_scatter(ctail, [pr, pc], tv, mask=m)
                plsc.store_scatter(ctype, [pos], yv, mask=m)
                return off + jnp.sum(m.astype(jnp.int32))

            n = lax.fori_loop(0, CH // 16, scan, jnp.int32(0))
            npos = n + iota
            nr_ = lax.shift_right_logical(npos, 4)
            ncl = npos & 15
            pad_row = jnp.full((16,), N_ENT, jnp.int32)
            zero_i = jnp.zeros((16,), jnp.int32)
            plsc.store_scatter(chabs, [nr_, ncl], pad_row)
            plsc.store_scatter(cdst, [nr_, ncl], zero_i)
            plsc.store_scatter(ctail, [nr_, ncl], pad_row)
            plsc.store_scatter(ctype, [npos], zero_i)
            ng = lax.shift_right_logical(n + 15, 4)

            def fire(g, b):
                pltpu.async_copy(ent_hbm.at[chabs.at[g]],
                                 hrows.at[pl.ds(b * 16, 16)], semh.at[b])
                pltpu.async_copy(ent_hbm.at[ctail.at[g]],
                                 trows.at[pl.ds(b * 16, 16)], semt.at[b])

            def drain(g, b):
                pltpu.make_async_copy(ent_hbm.at[chabs.at[g]],
                                      hrows.at[pl.ds(b * 16, 16)],
                                      semh.at[b]).wait()
                pltpu.make_async_copy(ent_hbm.at[ctail.at[g]],
                                      trows.at[pl.ds(b * 16, 16)],
                                      semt.at[b]).wait()

            for b0 in range(3):
                @pl.when(b0 < ng)
                def _(b0=b0):
                    fire(b0, b0)

            def quad(qi, _):
                gp = qi * 4
                for b in range(4):
                    g = gp + b

                    @pl.when(g < ng)
                    def _(g=g, b=b):
                        drain(g, b)

                        @pl.when(g + 3 < ng)
                        def _():
                            fire(g + 3, (b + 3) % 4)

                        ob = b & 1

                        @pl.when(g >= 2)
                        def _():
                            pltpu.make_async_copy(
                                orows.at[pl.ds(ob * 16, 16)],
                                acc.at[cdst.at[g - 2]], semo.at[ob]).wait()

                        gbase = g * 16
                        jbase = b * 16
                        for j in range(16):
                            ty = plsc.load_gather(
                                ctype,
                                [gbase + j + jnp.zeros((16,), jnp.int32)])
                            cqs = []
                            sacc = None
                            for q in range(NQ):
                                rq = plsc.load_gather(rtab,
                                                      [ty, iota + (q * 16)])
                                hq = hrows[jbase + j, pl.ds(q * 16, 16)]
                                tq = trows[jbase + j, pl.ds(q * 16, 16)]
                                cq = tq * rq
                                cqs.append(cq)
                                p = hq * cq
                                sacc = p if sacc is None else sacc + p
                            s = jnp.sum(sacc) * 0.125
                            esv = jnp.exp(jnp.full((16,), s, jnp.float32))
                            for q in range(NQ):
                                orows[ob * 16 + j, pl.ds(q * 16, 16)] = (
                                    esv * cqs[q])
                        pltpu.async_copy(orows.at[pl.ds(ob * 16, 16)],
                                         acc.at[cdst.at[g]], semo.at[ob],
                                         add=True)
                return 0

            lax.fori_loop(0, lax.shift_right_logical(ng + 3, 2), quad, 0)
            for back in (2, 1):
                @pl.when(ng >= back)
                def _(back=back):
                    gl = ng - back
                    obl = gl & 1
                    pltpu.make_async_copy(
                        orows.at[pl.ds(obl * 16, 16)],
                        acc.at[cdst.at[gl]], semo.at[obl]).wait()
            return 0

        lax.fori_loop(0, KG_NCH, chunk_body, 0)
        plsc.subcore_barrier()
        pltpu.sync_copy(
            acc.at[pl.ds(sid * KG_ROWS_TILE, KG_ROWS_TILE)],
            agg_hbm.at[pl.ds(lo + sid * KG_ROWS_TILE, KG_ROWS_TILE)])
        plsc.subcore_barrier()
        return 0

    lax.fori_loop(0, KG_R_PER_SC, range_body, 0)


@functools.partial(
    pl.kernel, mesh=_MESH,
    out_type=(jax.ShapeDtypeStruct((U_PAD, D), jnp.float32),
              jax.ShapeDtypeStruct((I_PAD, D), jnp.float32)),
    compiler_params=_SC_PARAMS,
    scratch_types=[
        pltpu.VMEM_SHARED((U_NR, D), jnp.float32),
        pltpu.VMEM((CHU,), jnp.int32),
        pltpu.VMEM((CHU,), jnp.int32),
        pltpu.VMEM((CHU,), jnp.float32),
        pltpu.VMEM((GRPU, 16), jnp.int32),
        pltpu.VMEM((GRPU, 16), jnp.int32),
        pltpu.VMEM((CHU + 16,), jnp.float32),
        pltpu.VMEM((4 * 16, D), jnp.float32),
        pltpu.VMEM((2 * 16, D), jnp.float32),
        pltpu.VMEM((16, D), jnp.float32),
        pltpu.SemaphoreType.DMA((4,)),
        pltpu.SemaphoreType.DMA((2,)),
    ])
def _ui_pass(usr_hbm, itm_hbm, iu_hbm, ii_hbm, w_hbm, uo_hbm, io_hbm,
             acc, dc, sc_, wc, cdst, csrc, cw, rows, orows, zbuf, semg, semo):
    cid = lax.axis_index("c")
    sid = lax.axis_index("s")
    iota = lax.iota(jnp.int32, 16)
    _zero_fill(zbuf)

    def side(dst_hbm_, src_hbm_, tab_hbm_, out_hbm_, nr_side, rows_tile):
        lo = cid * nr_side
        _zero_acc(acc, zbuf, sid, rows_tile)
        plsc.subcore_barrier()

        def chunk_body(c, _):
            base = sid * UI_SP + c * CHU
            c1 = pltpu.async_copy(dst_hbm_.at[pl.ds(base, CHU)], dc, semg.at[0])
            c2 = pltpu.async_copy(src_hbm_.at[pl.ds(base, CHU)], sc_, semg.at[1])
            c3 = pltpu.async_copy(w_hbm.at[pl.ds(base, CHU)], wc, semg.at[2])
            c1.wait()
            c2.wait()
            c3.wait()

            def scan(g, off):
                dv = dc[pl.ds(g * 16, 16)]
                sv = sc_[pl.ds(g * 16, 16)]
                wv = wc[pl.ds(g * 16, 16)]
                m = (dv >= lo) & (dv < lo + nr_side)
                pos = off + plsc.cumsum(m.astype(jnp.int32)) - 1
                pr = lax.shift_right_logical(pos, 4)
                pc = pos & 15
                plsc.store_scatter(cdst, [pr, pc], dv - lo, mask=m)
                plsc.store_scatter(csrc, [pr, pc], sv, mask=m)
                plsc.store_scatter(cw, [pos], wv, mask=m)
                return off + jnp.sum(m.astype(jnp.int32))

            n = lax.fori_loop(0, CHU // 16, scan, jnp.int32(0))
            npos = n + iota
            nr_ = lax.shift_right_logical(npos, 4)
            ncl = npos & 15
            zero_i = jnp.zeros((16,), jnp.int32)
            plsc.store_scatter(cdst, [nr_, ncl], zero_i)
            plsc.store_scatter(csrc, [nr_, ncl], zero_i)
            plsc.store_scatter(cw, [npos], jnp.zeros((16,), jnp.float32))
            ng = lax.shift_right_logical(n + 15, 4)

            def fire(g, b):
                pltpu.async_copy(tab_hbm_.at[csrc.at[g]],
                                 rows.at[pl.ds(b * 16, 16)], semg.at[b])

            def drain(g, b):
                pltpu.make_async_copy(tab_hbm_.at[csrc.at[g]],
                                      rows.at[pl.ds(b * 16, 16)],
                                      semg.at[b]).wait()

            for b0 in range(3):
                @pl.when(b0 < ng)
                def _(b0=b0):
                    fire(b0, b0)

            def quad(qi, _):
                gp = qi * 4
                for b in range(4):
                    g = gp + b

                    @pl.when(g < ng)
                    def _(g=g, b=b):
                        drain(g, b)

                        @pl.when(g + 3 < ng)
                        def _():
                            fire(g + 3, (b + 3) % 4)

                        ob = b & 1

                        @pl.when(g >= 2)
                        def _():
                            pltpu.make_async_copy(
                                orows.at[pl.ds(ob * 16, 16)],
                                acc.at[cdst.at[g - 2]], semo.at[ob]).wait()

                        gbase = g * 16
                        jbase = b * 16
                        for j in range(16):
                            wsp = plsc.load_gather(
                                cw, [gbase + j + jnp.zeros((16,), jnp.int32)])
                            for q in range(NQ):
                                orows[ob * 16 + j, pl.ds(q * 16, 16)] = (
                                    wsp * rows[jbase + j, pl.ds(q * 16, 16)])
                        pltpu.async_copy(orows.at[pl.ds(ob * 16, 16)],
                                         acc.at[cdst.at[g]], semo.at[ob],
                                         add=True)
                return 0

            lax.fori_loop(0, lax.shift_right_logical(ng + 3, 2), quad, 0)
            for back in (2, 1):
                @pl.when(ng >= back)
                def _(back=back):
                    gl = ng - back
                    obl = gl & 1
                    pltpu.make_async_copy(
                        orows.at[pl.ds(obl * 16, 16)],
                        acc.at[cdst.at[gl]], semo.at[obl]).wait()
            return 0

        lax.fori_loop(0, UI_NCH, chunk_body, 0)
        plsc.subcore_barrier()
        pltpu.sync_copy(
            acc.at[pl.ds(sid * rows_tile, rows_tile)],
            out_hbm_.at[pl.ds(lo + sid * rows_tile, rows_tile)])
        plsc.subcore_barrier()

    side(iu_hbm, ii_hbm, itm_hbm, uo_hbm, U_NR, U_ROWS_TILE)
    side(ii_hbm, iu_hbm, usr_hbm, io_hbm, I_NR, I_ROWS_TILE)


def _norm_body(agg_ref, resin_ref, ent_ref, resout_ref):
    x = agg_ref[...]
    ss = jnp.sum(x * x, axis=1, keepdims=True)
    ent = x / (jnp.sqrt(ss) + 1e-8)
    ent_ref[...] = ent
    resout_ref[...] = resin_ref[...] + ent


_NORM_BLK = 1024
_norm_call = pl.pallas_call(
    _norm_body,
    grid=(N_ENT_PAD // _NORM_BLK,),
    in_specs=[pl.BlockSpec((_NORM_BLK, D), lambda i: (i, 0))] * 2,
    out_specs=[pl.BlockSpec((_NORM_BLK, D), lambda i: (i, 0))] * 2,
    out_shape=(jax.ShapeDtypeStruct((N_ENT_PAD, D), jnp.float32),
               jax.ShapeDtypeStruct((N_ENT_PAD, D), jnp.float32)),
)


def _add3_body(a_ref, b_ref, c_ref, o_ref):
    o_ref[...] = a_ref[...] + b_ref[...] + c_ref[...]


def _add3(a, b, c, blk):
    n = a.shape[0]
    return pl.pallas_call(
        _add3_body,
        grid=(n // blk,),
        in_specs=[pl.BlockSpec((blk, D), lambda i: (i, 0))] * 3,
        out_specs=pl.BlockSpec((blk, D), lambda i: (i, 0)),
        out_shape=jax.ShapeDtypeStruct((n, D), jnp.float32),
    )(a, b, c)


def kernel(entity_emb, user_emb, relation_emb, edge_index, edge_type,
           inter_user, inter_item, inter_edge_w):
    f32 = jnp.float32
    i32 = jnp.int32
    head = edge_index[0].astype(i32)
    tail = edge_index[1].astype(i32)
    etype = edge_type.astype(i32)

    ent0 = jnp.zeros((N_ENT_PAD, D), f32).at[:N_ENT].set(entity_emb)
    headp = jnp.full((E_KG_PAD,), -1, i32).at[:E_KG].set(head)
    tailp = jnp.zeros((E_KG_PAD,), i32).at[:E_KG].set(tail)
    typep = jnp.zeros((E_KG_PAD,), i32).at[:E_KG].set(etype)
    iup = jnp.full((E_UI_PAD,), -1, i32).at[:E_UI].set(inter_user.astype(i32))
    iip = jnp.full((E_UI_PAD,), -1, i32).at[:E_UI].set(inter_item.astype(i32))
    wp = jnp.zeros((E_UI_PAD,), f32).at[:E_UI].set(inter_edge_w)

    agg1 = _kg_pass(ent0, relation_emb, headp, tailp, typep)
    ent1, res1 = _norm_call(agg1, ent0)
    agg2 = _kg_pass(ent1, relation_emb, headp, tailp, typep)
    _, res2 = _norm_call(agg2, res1)
    ent_res = res2[:N_ENT]

    u0 = user_emb
    i0 = entity_emb[:N_ITM]
    u1p, i1p = _ui_pass(u0, i0, iup, iip, wp)
    u1 = u1p[:N_USR]
    i1 = i1p[:N_ITM]
    u2p, i2p = _ui_pass(u1, i1, iup, iip, wp)
    u_res = _add3(u0, u1, u2p[:N_USR], 1000)
    i_res = _add3(i0, i1, i2p[:N_ITM], 1000)
    return ent_res, u_res, i_res
